# baseline (speedup 1.0000x reference)
import jax
import jax.numpy as jnp
from jax.experimental import pallas as pl
from jax.experimental.pallas import tpu as pltpu

S = 4
NB = 2
IN = 256
OUT = 256
N = 4096
BR = 256


def _rgc_body(adj_ref, x_ref, bp_ref, cp_ref, bias_ref, out_ref, y_ref):
    i = pl.program_id(0)
    s = pl.program_id(1)

    def do(sc):
        @pl.when(i == 0)
        def _():
            v = (cp_ref[0][:, None] * bp_ref[0, 0]
                 + cp_ref[1][:, None] * bp_ref[0, 1])
            y = jnp.dot(x_ref[:], v.astype(jnp.bfloat16),
                        preferred_element_type=jnp.float32)
            y_ref[sc] = y.astype(jnp.bfloat16)

        a_bf = adj_ref[:].astype(jnp.bfloat16)
        contrib = jnp.dot(a_bf, y_ref[sc], preferred_element_type=jnp.float32)
        if sc == 0:
            out_ref[:] = contrib + bias_ref[:]
        else:
            out_ref[:] = out_ref[:] + contrib

    for sc in range(S):
        pl.when(s == sc)(lambda sc=sc: do(sc))


def kernel(input, adjs, basis, coef, bias):
    basis_r = basis.reshape(NB, IN, OUT)
    f = jnp.arange(IN)
    rows = jnp.arange(S)[:, None] * (IN // S) + (f // S)[None, :]
    bp = jnp.transpose(basis_r[:, rows, :], (1, 0, 2, 3))
    cp = coef[f % S, :].T
    bias2 = bias.reshape(1, OUT)
    xb = input.astype(jnp.bfloat16)
    adjs2 = adjs.reshape(S * N, N)

    grid = (N // BR, S)
    nblk = N // BR
    out = pl.pallas_call(
        _rgc_body,
        grid=grid,
        in_specs=[
            pl.BlockSpec((BR, N), lambda i, s: (s * nblk + i, 0)),
            pl.BlockSpec((N, IN), lambda i, s: (0, 0)),
            pl.BlockSpec((1, NB, IN, OUT), lambda i, s: (s, 0, 0, 0)),
            pl.BlockSpec((NB, IN), lambda i, s: (0, 0)),
            pl.BlockSpec((1, OUT), lambda i, s: (0, 0)),
        ],
        out_specs=pl.BlockSpec((BR, OUT), lambda i, s: (i, 0)),
        out_shape=jax.ShapeDtypeStruct((N, OUT), jnp.float32),
        scratch_shapes=[pltpu.VMEM((S, N, OUT), jnp.bfloat16)],
        compiler_params=pltpu.CompilerParams(
            dimension_semantics=("parallel", "arbitrary")),
    )(adjs2, xb, bp, cp, bias2)
    return out


# speedup vs baseline: 1.0032x; 1.0032x over previous
import jax
import jax.numpy as jnp
from jax.experimental import pallas as pl
from jax.experimental.pallas import tpu as pltpu

S = 4
NB = 2
IN = 256
OUT = 256
N = 4096
BR = 256
NQ = 4
CQ = N // NQ


def _rgc_body(a0_ref, a1_ref, a2_ref, a3_ref, x_ref, bp_ref, cp_ref,
              bias_ref, out_ref, y_ref):
    i = pl.program_id(0)
    s = pl.program_id(1)
    a_refs = (a0_ref, a1_ref, a2_ref, a3_ref)

    def do(sc):
        @pl.when(i == 0)
        def _():
            v = (cp_ref[0][:, None] * bp_ref[0, 0]
                 + cp_ref[1][:, None] * bp_ref[0, 1])
            y = jnp.dot(x_ref[:], v.astype(jnp.bfloat16),
                        preferred_element_type=jnp.float32)
            y_ref[sc] = y.astype(jnp.bfloat16)

        contrib = None
        for q in range(NQ):
            a_bf = a_refs[q][:].astype(jnp.bfloat16)
            p = jnp.dot(a_bf, y_ref[sc, q * CQ:(q + 1) * CQ, :],
                        preferred_element_type=jnp.float32)
            contrib = p if contrib is None else contrib + p
        if sc == 0:
            out_ref[:] = contrib + bias_ref[:]
        else:
            out_ref[:] = out_ref[:] + contrib

    for sc in range(S):
        pl.when(s == sc)(lambda sc=sc: do(sc))


def kernel(input, adjs, basis, coef, bias):
    basis_r = basis.reshape(NB, IN, OUT)
    f = jnp.arange(IN)
    rows = jnp.arange(S)[:, None] * (IN // S) + (f // S)[None, :]
    bp = jnp.transpose(basis_r[:, rows, :], (1, 0, 2, 3))
    cp = coef[f % S, :].T
    bias2 = bias.reshape(1, OUT)
    xb = input.astype(jnp.bfloat16)
    adjs2 = adjs.reshape(S * N, N)

    grid = (N // BR, S)
    nblk = N // BR
    adj_specs = [
        pl.BlockSpec((BR, CQ), lambda i, s, q=q: (s * nblk + i, q))
        for q in range(NQ)
    ]
    out = pl.pallas_call(
        _rgc_body,
        grid=grid,
        in_specs=adj_specs + [
            pl.BlockSpec((N, IN), lambda i, s: (0, 0)),
            pl.BlockSpec((1, NB, IN, OUT), lambda i, s: (s, 0, 0, 0)),
            pl.BlockSpec((NB, IN), lambda i, s: (0, 0)),
            pl.BlockSpec((1, OUT), lambda i, s: (0, 0)),
        ],
        out_specs=pl.BlockSpec((BR, OUT), lambda i, s: (i, 0)),
        out_shape=jax.ShapeDtypeStruct((N, OUT), jnp.float32),
        scratch_shapes=[pltpu.VMEM((S, N, OUT), jnp.bfloat16)],
        compiler_params=pltpu.CompilerParams(
            dimension_semantics=("parallel", "arbitrary")),
    )(adjs2, adjs2, adjs2, adjs2, xb, bp, cp, bias2)
    return out


# speedup vs baseline: 1.2034x; 1.1996x over previous
import jax
import jax.numpy as jnp
from jax.experimental import pallas as pl
from jax.experimental.pallas import tpu as pltpu

S = 4
NB = 2
IN = 256
OUT = 256
N = 4096
BR = 512
NQ = 4
CQ = N // NQ


def _rgc_body(a0_ref, a1_ref, a2_ref, a3_ref, x_ref, bp_ref, cp_ref,
              bias_ref, out_ref, y_ref):
    i = pl.program_id(0)
    s = pl.program_id(1)
    a_refs = (a0_ref, a1_ref, a2_ref, a3_ref)

    def do(sc):
        @pl.when(i == 0)
        def _():
            v = (cp_ref[0][:, None] * bp_ref[0, 0]
                 + cp_ref[1][:, None] * bp_ref[0, 1])
            y = jnp.dot(x_ref[:], v.astype(jnp.bfloat16),
                        preferred_element_type=jnp.float32)
            y_ref[sc] = y.astype(jnp.bfloat16)

        contrib = None
        for q in range(NQ):
            a_bf = a_refs[q][:].astype(jnp.bfloat16)
            p = jnp.dot(a_bf, y_ref[sc, q * CQ:(q + 1) * CQ, :],
                        preferred_element_type=jnp.float32)
            contrib = p if contrib is None else contrib + p
        if sc == 0:
            out_ref[:] = contrib + bias_ref[:]
        else:
            out_ref[:] = out_ref[:] + contrib

    for sc in range(S):
        pl.when(s == sc)(lambda sc=sc: do(sc))


def kernel(input, adjs, basis, coef, bias):
    basis_r = basis.reshape(NB, IN, OUT)
    f = jnp.arange(IN)
    rows = jnp.arange(S)[:, None] * (IN // S) + (f // S)[None, :]
    bp = jnp.transpose(basis_r[:, rows, :], (1, 0, 2, 3))
    cp = coef[f % S, :].T
    bias2 = bias.reshape(1, OUT)
    xb = input.astype(jnp.bfloat16)
    adjs2 = adjs.reshape(S * N, N)

    grid = (N // BR, S)
    nblk = N // BR
    adj_specs = [
        pl.BlockSpec((BR, CQ), lambda i, s, q=q: (s * nblk + i, q))
        for q in range(NQ)
    ]
    out = pl.pallas_call(
        _rgc_body,
        grid=grid,
        in_specs=adj_specs + [
            pl.BlockSpec((N, IN), lambda i, s: (0, 0)),
            pl.BlockSpec((1, NB, IN, OUT), lambda i, s: (s, 0, 0, 0)),
            pl.BlockSpec((NB, IN), lambda i, s: (0, 0)),
            pl.BlockSpec((1, OUT), lambda i, s: (0, 0)),
        ],
        out_specs=pl.BlockSpec((BR, OUT), lambda i, s: (i, 0)),
        out_shape=jax.ShapeDtypeStruct((N, OUT), jnp.float32),
        scratch_shapes=[pltpu.VMEM((S, N, OUT), jnp.bfloat16)],
        compiler_params=pltpu.CompilerParams(
            dimension_semantics=("parallel", "arbitrary")),
    )(adjs2, adjs2, adjs2, adjs2, xb, bp, cp, bias2)
    return out


# speedup vs baseline: 1.2244x; 1.0175x over previous
import jax
import jax.numpy as jnp
from jax.experimental import pallas as pl
from jax.experimental.pallas import tpu as pltpu

S = 4
NB = 2
IN = 256
OUT = 256
N = 4096
BR = 1024
NQ = 4
CQ = N // NQ


def _rgc_body(a0_ref, a1_ref, a2_ref, a3_ref, x_ref, bp_ref, cp_ref,
              bias_ref, out_ref, y_ref):
    i = pl.program_id(0)
    s = pl.program_id(1)
    a_refs = (a0_ref, a1_ref, a2_ref, a3_ref)

    def do(sc):
        @pl.when(i == 0)
        def _():
            v = (cp_ref[0][:, None] * bp_ref[0, 0]
                 + cp_ref[1][:, None] * bp_ref[0, 1])
            y = jnp.dot(x_ref[:], v.astype(jnp.bfloat16),
                        preferred_element_type=jnp.float32)
            y_ref[sc] = y.astype(jnp.bfloat16)

        contrib = None
        for q in range(NQ):
            a_bf = a_refs[q][:].astype(jnp.bfloat16)
            p = jnp.dot(a_bf, y_ref[sc, q * CQ:(q + 1) * CQ, :],
                        preferred_element_type=jnp.float32)
            contrib = p if contrib is None else contrib + p
        if sc == 0:
            out_ref[:] = contrib + bias_ref[:]
        else:
            out_ref[:] = out_ref[:] + contrib

    for sc in range(S):
        pl.when(s == sc)(lambda sc=sc: do(sc))


def kernel(input, adjs, basis, coef, bias):
    basis_r = basis.reshape(NB, IN, OUT)
    f = jnp.arange(IN)
    rows = jnp.arange(S)[:, None] * (IN // S) + (f // S)[None, :]
    bp = jnp.transpose(basis_r[:, rows, :], (1, 0, 2, 3))
    cp = coef[f % S, :].T
    bias2 = bias.reshape(1, OUT)
    xb = input.astype(jnp.bfloat16)
    adjs2 = adjs.reshape(S * N, N)

    grid = (N // BR, S)
    nblk = N // BR
    adj_specs = [
        pl.BlockSpec((BR, CQ), lambda i, s, q=q: (s * nblk + i, q))
        for q in range(NQ)
    ]
    out = pl.pallas_call(
        _rgc_body,
        grid=grid,
        in_specs=adj_specs + [
            pl.BlockSpec((N, IN), lambda i, s: (0, 0)),
            pl.BlockSpec((1, NB, IN, OUT), lambda i, s: (s, 0, 0, 0)),
            pl.BlockSpec((NB, IN), lambda i, s: (0, 0)),
            pl.BlockSpec((1, OUT), lambda i, s: (0, 0)),
        ],
        out_specs=pl.BlockSpec((BR, OUT), lambda i, s: (i, 0)),
        out_shape=jax.ShapeDtypeStruct((N, OUT), jnp.float32),
        scratch_shapes=[pltpu.VMEM((S, N, OUT), jnp.bfloat16)],
        compiler_params=pltpu.CompilerParams(
            dimension_semantics=("parallel", "arbitrary")),
    )(adjs2, adjs2, adjs2, adjs2, xb, bp, cp, bias2)
    return out
